# baseline (speedup 1.0000x reference)
import jax
import jax.numpy as jnp
from jax.experimental import pallas as pl
from jax.experimental.pallas import tpu as pltpu

_BN_EPS = 1e-5
_NORM_EPS = 1e-12
_VMEM_LIMIT = 48 * 1024 * 1024


def _conv_gap_body(g_ref, w_ref, o_ref, *, img_tile, h_img, w_img):
    hw = h_img * w_img
    w = w_ref[...]
    inv = 1.0 / hw
    for i in range(img_tile):
        p = jnp.concatenate(
            [g_ref[i, 0:hw, :],
             g_ref[i, w_img:w_img + hw, 0:9],
             g_ref[i, 2 * w_img:2 * w_img + hw, 0:9]],
            axis=1)
        y = jnp.dot(p, w, preferred_element_type=jnp.float32)
        y = jnp.maximum(y, 0.0)
        s = jnp.sum(y, axis=0, keepdims=True)
        o_ref[pl.ds(i, 1), :] = (s * inv).astype(o_ref.dtype)


def _conv_gap(g, w_ext, *, h_img, w_img, f_dim, img_tile=8):
    BB, rows, L = g.shape
    body = lambda gr, wr, o: _conv_gap_body(
        gr, wr, o, img_tile=img_tile, h_img=h_img, w_img=w_img)
    return pl.pallas_call(
        body,
        out_shape=jax.ShapeDtypeStruct((BB, f_dim), jnp.bfloat16),
        grid=(BB // img_tile,),
        in_specs=[
            pl.BlockSpec((img_tile, rows, L), lambda b: (b, 0, 0)),
            pl.BlockSpec(w_ext.shape, lambda b: (0, 0)),
        ],
        out_specs=pl.BlockSpec((img_tile, f_dim), lambda b: (b, 0)),
        compiler_params=pltpu.CompilerParams(
            dimension_semantics=("parallel",),
            vmem_limit_bytes=_VMEM_LIMIT),
    )(g, w_ext)


def _heads_body(f1, f2,
                ow1, ob1, og, obt, ow2, ob2,
                pw1, pb1, pg, pbt, pw2, pb2,
                tw1, tb1, tg, tbt, tw2, tb2,
                o_ref):
    def head(x, w1, b1, g, bt, w2, b2):
        pre = jnp.dot(x, w1[...], preferred_element_type=jnp.float32) + b1[...]
        mu = jnp.mean(pre, axis=0, keepdims=True)
        d = pre - mu
        var = jnp.mean(d * d, axis=0, keepdims=True)
        act = jnp.maximum(d * jax.lax.rsqrt(var + _BN_EPS) * g[...] + bt[...],
                          0.0)
        return jnp.dot(act.astype(w2.dtype), w2[...],
                       preferred_element_type=jnp.float32) + b2[...]

    z1 = head(f1[...], ow1, ob1, og, obt, ow2, ob2)
    z2 = head(f2[...], tw1, tb1, tg, tbt, tw2, tb2)
    q = head(z1.astype(pw1.dtype), pw1, pb1, pg, pbt, pw2, pb2)
    inv1 = jax.lax.rsqrt(jnp.maximum(
        jnp.sum(q * q, axis=-1, keepdims=True), _NORM_EPS * _NORM_EPS))
    inv2 = jax.lax.rsqrt(jnp.maximum(
        jnp.sum(z2 * z2, axis=-1, keepdims=True), _NORM_EPS * _NORM_EPS))
    sim = jnp.sum((q * inv1) * (z2 * inv2), axis=-1)
    o_ref[0] = 2.0 - 2.0 * (jnp.sum(sim) / q.shape[0])


def _heads_loss(f1, f2, args):
    def full(a):
        nd = a.ndim
        return pl.BlockSpec(a.shape, lambda _nd=nd: (0,) * _nd)

    ops = [f1, f2] + list(args)
    out = pl.pallas_call(
        _heads_body,
        out_shape=jax.ShapeDtypeStruct((1,), jnp.float32),
        grid=(),
        in_specs=[full(a) for a in ops],
        out_specs=pl.BlockSpec(memory_space=pltpu.MemorySpace.SMEM),
        compiler_params=pltpu.CompilerParams(
            vmem_limit_bytes=_VMEM_LIMIT),
    )(*ops)
    return out[0]


def _dw_expand(x_nchw):
    x = jnp.transpose(x_nchw, (0, 2, 3, 1)).astype(jnp.bfloat16)
    B, H, W, C = x.shape
    xp = jnp.pad(x, ((0, 0), (1, 1), (1, 1), (0, 0)))
    crops = [xp[:, :, dw:dw + W, :] for dw in range(3)]
    crops.append(jnp.ones((B, H + 2, W, 2), jnp.bfloat16))
    g = jnp.concatenate(crops, axis=-1)
    return g.reshape(B, (H + 2) * W, 3 * C + 2)


def kernel(x1, x2, conv_w, conv_b,
           on_w1, on_b1, on_gamma, on_beta, on_w2, on_b2,
           pr_w1, pr_b1, pr_gamma, pr_beta, pr_w2, pr_b2,
           tg_w1, tg_b1, tg_gamma, tg_beta, tg_w2, tg_b2):
    B = x1.shape[0]
    H, W = x1.shape[2], x1.shape[3]
    F = conv_w.shape[1]
    g = jnp.concatenate([_dw_expand(x1), _dw_expand(x2)], axis=0)
    b_hi = conv_b.astype(jnp.bfloat16)
    b_lo = (conv_b - b_hi.astype(jnp.float32)).astype(jnp.bfloat16)
    w_ext = jnp.concatenate(
        [conv_w[0:9], b_hi, b_lo, conv_w[9:18], conv_w[18:27]], axis=0)

    f = _conv_gap(g, w_ext, h_img=H, w_img=W, f_dim=F)
    f1, f2 = f[:B], f[B:]
    return _heads_loss(f1, f2, [
        on_w1, on_b1, on_gamma, on_beta, on_w2, on_b2,
        pr_w1, pr_b1, pr_gamma, pr_beta, pr_w2, pr_b2,
        tg_w1, tg_b1, tg_gamma, tg_beta, tg_w2, tg_b2])


# speedup vs baseline: 1.3162x; 1.3162x over previous
import jax
import jax.numpy as jnp
from jax.experimental import pallas as pl
from jax.experimental.pallas import tpu as pltpu

_BN_EPS = 1e-5
_NORM_EPS = 1e-12
_VMEM_LIMIT = 48 * 1024 * 1024


_Q = 4


def _conv_gap_body(g_ref, w_ref, o_ref, *, img_tile, h_img, w_img, f_dim):
    wq = w_img // _Q
    hwq = h_img * wq
    w = w_ref[...]
    inv = 1.0 / (h_img * w_img)
    for i in range(img_tile):
        p = jnp.concatenate(
            [g_ref[i, 0:hwq, :],
             g_ref[i, wq:wq + hwq, 0:18],
             g_ref[i, 2 * wq:2 * wq + hwq, 0:18]],
            axis=1)
        y = jnp.dot(p, w, preferred_element_type=jnp.float32)
        y = jnp.maximum(y, 0.0)
        s = jnp.sum(y, axis=0, keepdims=True)
        s = sum(s[:, k * f_dim:(k + 1) * f_dim] for k in range(_Q))
        o_ref[pl.ds(i, 1), :] = (s * inv).astype(o_ref.dtype)


def _conv_gap(g, w4, *, h_img, w_img, f_dim, img_tile=8):
    BB, rows, L = g.shape
    body = lambda gr, wr, o: _conv_gap_body(
        gr, wr, o, img_tile=img_tile, h_img=h_img, w_img=w_img, f_dim=f_dim)
    return pl.pallas_call(
        body,
        out_shape=jax.ShapeDtypeStruct((BB, f_dim), jnp.bfloat16),
        grid=(BB // img_tile,),
        in_specs=[
            pl.BlockSpec((img_tile, rows, L), lambda b: (b, 0, 0)),
            pl.BlockSpec(w4.shape, lambda b: (0, 0)),
        ],
        out_specs=pl.BlockSpec((img_tile, f_dim), lambda b: (b, 0)),
        compiler_params=pltpu.CompilerParams(
            dimension_semantics=("parallel",),
            vmem_limit_bytes=_VMEM_LIMIT),
    )(g, w4)


def _heads_body(f1, f2,
                ow1, ob1, og, obt, ow2, ob2,
                pw1, pb1, pg, pbt, pw2, pb2,
                tw1, tb1, tg, tbt, tw2, tb2,
                o_ref):
    def head(x, w1, b1, g, bt, w2, b2):
        pre = jnp.dot(x, w1[...], preferred_element_type=jnp.float32) + b1[...]
        mu = jnp.mean(pre, axis=0, keepdims=True)
        d = pre - mu
        var = jnp.mean(d * d, axis=0, keepdims=True)
        act = jnp.maximum(d * jax.lax.rsqrt(var + _BN_EPS) * g[...] + bt[...],
                          0.0)
        return jnp.dot(act.astype(w2.dtype), w2[...],
                       preferred_element_type=jnp.float32) + b2[...]

    z1 = head(f1[...], ow1, ob1, og, obt, ow2, ob2)
    z2 = head(f2[...], tw1, tb1, tg, tbt, tw2, tb2)
    q = head(z1.astype(pw1.dtype), pw1, pb1, pg, pbt, pw2, pb2)
    inv1 = jax.lax.rsqrt(jnp.maximum(
        jnp.sum(q * q, axis=-1, keepdims=True), _NORM_EPS * _NORM_EPS))
    inv2 = jax.lax.rsqrt(jnp.maximum(
        jnp.sum(z2 * z2, axis=-1, keepdims=True), _NORM_EPS * _NORM_EPS))
    sim = jnp.sum((q * inv1) * (z2 * inv2), axis=-1)
    o_ref[0] = 2.0 - 2.0 * (jnp.sum(sim) / q.shape[0])


def _heads_loss(f1, f2, args):
    def full(a):
        nd = a.ndim
        return pl.BlockSpec(a.shape, lambda _nd=nd: (0,) * _nd)

    ops = [f1, f2] + list(args)
    out = pl.pallas_call(
        _heads_body,
        out_shape=jax.ShapeDtypeStruct((1,), jnp.float32),
        grid=(),
        in_specs=[full(a) for a in ops],
        out_specs=pl.BlockSpec(memory_space=pltpu.MemorySpace.SMEM),
        compiler_params=pltpu.CompilerParams(
            vmem_limit_bytes=_VMEM_LIMIT),
    )(*ops)
    return out[0]


def _quad_expand(x_nchw):
    x = jnp.transpose(x_nchw.astype(jnp.bfloat16), (0, 2, 3, 1))
    B, H, W, C = x.shape
    wq = W // _Q
    xp = jnp.pad(x, ((0, 0), (1, 1), (1, 3), (0, 0)))
    r = xp.reshape(B, H + 2, wq + 1, _Q, C)
    crops = [r[:, :, 0:wq, dw, :] for dw in range(_Q)]
    crops += [r[:, :, 1:wq + 1, dw, :] for dw in range(2)]
    crops.append(jnp.ones((B, H + 2, wq, 2), jnp.bfloat16))
    g = jnp.concatenate(crops, axis=-1)
    return g.reshape(B, (H + 2) * wq, 6 * C + 2)


def _quad_weights(conv_w, conv_b, f_dim):
    W = conv_w.reshape(3, 3, 3, f_dim)
    zero = jnp.zeros((3, f_dim), conv_w.dtype)
    b_hi = conv_b.astype(jnp.bfloat16)
    b_lo = (conv_b - b_hi.astype(jnp.float32)).astype(jnp.bfloat16)
    blocks = []
    for dh in range(3):
        cols = []
        for col in range(6):
            per_wi = [W[dh, col - wi] if 0 <= col - wi < 3 else zero
                      for wi in range(_Q)]
            cols.append(jnp.stack(per_wi, axis=1))
        blk = jnp.stack(cols, axis=0).reshape(18, _Q * f_dim)
        if dh == 0:
            bias = jnp.concatenate([jnp.tile(b_hi, (1, _Q)),
                                    jnp.tile(b_lo, (1, _Q))], axis=0)
            blk = jnp.concatenate([blk, bias], axis=0)
        blocks.append(blk)
    return jnp.concatenate(blocks, axis=0)


def kernel(x1, x2, conv_w, conv_b,
           on_w1, on_b1, on_gamma, on_beta, on_w2, on_b2,
           pr_w1, pr_b1, pr_gamma, pr_beta, pr_w2, pr_b2,
           tg_w1, tg_b1, tg_gamma, tg_beta, tg_w2, tg_b2):
    B = x1.shape[0]
    H, W = x1.shape[2], x1.shape[3]
    F = conv_w.shape[1]
    g = _quad_expand(jnp.concatenate(
        [x1.astype(jnp.bfloat16), x2.astype(jnp.bfloat16)], axis=0))
    w4 = _quad_weights(conv_w, conv_b, F)

    f = _conv_gap(g, w4, h_img=H, w_img=W, f_dim=F)
    f1, f2 = f[:B], f[B:]
    return _heads_loss(f1, f2, [
        on_w1, on_b1, on_gamma, on_beta, on_w2, on_b2,
        pr_w1, pr_b1, pr_gamma, pr_beta, pr_w2, pr_b2,
        tg_w1, tg_b1, tg_gamma, tg_beta, tg_w2, tg_b2])
